# baseline (speedup 1.0000x reference)
import functools

import jax
import jax.numpy as jnp
from jax import lax
from jax.experimental import pallas as pl
from jax.experimental.pallas import tpu as pltpu
from jax.experimental.pallas import tpu_sc as plsc

N_TOK_BLK = 2048
NEG_INF = float("-inf")


def _gemm_body(x_ref, w_ref, o_ref):
    o_ref[...] = jax.lax.dot_general(
        x_ref[...], w_ref[...], (((1,), (1,)), ((), ())),
        preferred_element_type=jnp.float32,
    )


def _tc_logits(x, W):
    n, d = x.shape
    num_e = W.shape[0]
    return pl.pallas_call(
        _gemm_body,
        grid=(n // N_TOK_BLK,),
        in_specs=[
            pl.BlockSpec((N_TOK_BLK, d), lambda t: (t, 0)),
            pl.BlockSpec((num_e, d), lambda t: (0, 0)),
        ],
        out_specs=pl.BlockSpec((N_TOK_BLK, num_e), lambda t: (t, 0)),
        out_shape=jax.ShapeDtypeStruct((n, num_e), jnp.float32),
        compiler_params=pltpu.CompilerParams(
            dimension_semantics=("arbitrary",),
        ),
    )(x, W)


def _make_sc_top2(n, num_e):
    info = plsc.get_sparse_core_info()
    nc, ns, lanes = info.num_cores, info.num_subcores, info.num_lanes
    nw = nc * ns
    tpw = n // nw
    nvec = num_e // lanes
    mesh = plsc.VectorSubcoreMesh(core_axis_name="c", subcore_axis_name="s")

    @functools.partial(
        pl.kernel,
        mesh=mesh,
        out_type=[
            jax.ShapeDtypeStruct((n,), jnp.float32),
            jax.ShapeDtypeStruct((n,), jnp.float32),
            jax.ShapeDtypeStruct((n,), jnp.int32),
            jax.ShapeDtypeStruct((n,), jnp.int32),
        ],
        scratch_types=[
            pltpu.VMEM((tpw, num_e), jnp.float32),
            pltpu.VMEM((tpw,), jnp.float32),
            pltpu.VMEM((tpw,), jnp.float32),
            pltpu.VMEM((tpw,), jnp.int32),
            pltpu.VMEM((tpw,), jnp.int32),
        ],
    )
    def sc_top2(lg_hbm, g1_hbm, g2_hbm, i1_hbm, i2_hbm,
                lg_v, g1_v, g2_v, i1_v, i2_v):
        wid = lax.axis_index("s") * nc + lax.axis_index("c")
        base = wid * tpw
        pltpu.sync_copy(lg_hbm.at[pl.ds(base, tpw), :], lg_v)

        iota = lax.iota(jnp.int32, lanes)
        idxs = [iota + j * lanes for j in range(nvec)]

        def shuf(v, idx):
            return lax.gather(
                v, idx[:, None],
                lax.GatherDimensionNumbers(
                    offset_dims=(), collapsed_slice_dims=(0,),
                    start_index_map=(0,)),
                slice_sizes=(1,),
                mode=lax.GatherScatterMode.PROMISE_IN_BOUNDS)

        def allreduce(v, op):
            for s in (8, 4, 2, 1):
                v = op(v, shuf(v, iota ^ s))
            return v

        def group_body(g, carry):
            a_m1 = jnp.zeros((lanes,), jnp.float32)
            a_m2 = jnp.zeros((lanes,), jnp.float32)
            a_i1 = jnp.zeros((lanes,), jnp.int32)
            a_i2 = jnp.zeros((lanes,), jnp.int32)
            for k in range(lanes):
                tok = g * lanes + k
                vs = [lg_v[tok, pl.ds(j * lanes, lanes)] for j in range(nvec)]
                m = vs[0]
                for j in range(1, nvec):
                    m = jnp.maximum(m, vs[j])
                m1 = allreduce(m, jnp.maximum)
                cand = jnp.where(vs[0] == m1, idxs[0], num_e)
                for j in range(1, nvec):
                    cand = jnp.minimum(
                        cand, jnp.where(vs[j] == m1, idxs[j], num_e))
                i1 = allreduce(cand, jnp.minimum)
                ms = [jnp.where(idxs[j] == i1, NEG_INF, vs[j])
                      for j in range(nvec)]
                m2v = ms[0]
                for j in range(1, nvec):
                    m2v = jnp.maximum(m2v, ms[j])
                m2 = allreduce(m2v, jnp.maximum)
                cand2 = jnp.where(ms[0] == m2, idxs[0], num_e)
                for j in range(1, nvec):
                    cand2 = jnp.minimum(
                        cand2, jnp.where(ms[j] == m2, idxs[j], num_e))
                i2 = allreduce(cand2, jnp.minimum)
                lane = iota == k
                a_m1 = jnp.where(lane, m1, a_m1)
                a_m2 = jnp.where(lane, m2, a_m2)
                a_i1 = jnp.where(lane, i1, a_i1)
                a_i2 = jnp.where(lane, i2, a_i2)
            t = jnp.exp(a_m2 - a_m1)
            g1 = 1.0 / (1.0 + t)
            row = pl.ds(g * lanes, lanes)
            g1_v[row] = g1
            g2_v[row] = t * g1
            i1_v[row] = a_i1
            i2_v[row] = a_i2
            return carry

        lax.fori_loop(0, tpw // lanes, group_body, 0)

        pltpu.sync_copy(g1_v, g1_hbm.at[pl.ds(base, tpw)])
        pltpu.sync_copy(g2_v, g2_hbm.at[pl.ds(base, tpw)])
        pltpu.sync_copy(i1_v, i1_hbm.at[pl.ds(base, tpw)])
        pltpu.sync_copy(i2_v, i2_hbm.at[pl.ds(base, tpw)])

    return sc_top2


@jax.jit
def _router(x, W):
    n, _ = x.shape
    num_e = W.shape[0]
    logits = _tc_logits(x, W)
    g1, g2, i1, i2 = _make_sc_top2(n, num_e)(logits)
    gates = jnp.stack([g1, g2], axis=-1)
    idx = jnp.stack([i1, i2], axis=-1)
    return gates, idx


def kernel(x, W):
    gates, idx = _router(x, W)
    return gates, idx, jnp.zeros((), dtype=jnp.float32)


# speedup vs baseline: 1.3329x; 1.3329x over previous
import jax
import jax.numpy as jnp
from jax.experimental import pallas as pl
from jax.experimental.pallas import tpu as pltpu

N_TOK_BLK = 2048


def _router_body(x_ref, w_ref, g_ref, i_ref):
    xb = x_ref[...]
    w = w_ref[...]
    logits = jax.lax.dot_general(
        xb, w, (((1,), (1,)), ((), ())), preferred_element_type=jnp.float32
    )
    e = logits.shape[-1]
    iota = jax.lax.broadcasted_iota(jnp.int32, logits.shape, 1).astype(
        jnp.float32)
    ef = jnp.float32(e)
    m1 = jnp.max(logits, axis=-1, keepdims=True)
    i1 = jnp.min(jnp.where(logits == m1, iota, ef), axis=-1, keepdims=True)
    masked = jnp.where(iota == i1, -jnp.inf, logits)
    m2 = jnp.max(masked, axis=-1, keepdims=True)
    i2 = jnp.min(jnp.where(masked == m2, iota, ef), axis=-1, keepdims=True)
    t = jnp.exp(m2 - m1)
    g1 = 1.0 / (1.0 + t)
    g2 = t * g1
    g_ref[...] = jnp.concatenate([g1, g2], axis=-1)
    i_ref[...] = jnp.concatenate([i1, i2], axis=-1).astype(jnp.int32)


@jax.jit
def _router(x, W):
    n, d = x.shape
    num_e = W.shape[0]
    grid = (n // N_TOK_BLK,)
    gates, idx = pl.pallas_call(
        _router_body,
        grid=grid,
        in_specs=[
            pl.BlockSpec((N_TOK_BLK, d), lambda t: (t, 0)),
            pl.BlockSpec((num_e, d), lambda t: (0, 0)),
        ],
        out_specs=[
            pl.BlockSpec((N_TOK_BLK, 2), lambda t: (t, 0)),
            pl.BlockSpec((N_TOK_BLK, 2), lambda t: (t, 0)),
        ],
        out_shape=[
            jax.ShapeDtypeStruct((n, 2), jnp.float32),
            jax.ShapeDtypeStruct((n, 2), jnp.int32),
        ],
        compiler_params=pltpu.CompilerParams(
            dimension_semantics=("arbitrary",),
        ),
    )(x, W)
    return gates, idx


def kernel(x, W):
    gates, idx = _router(x, W)
    return gates, idx, jnp.zeros((), dtype=jnp.float32)
